# baseline (speedup 1.0000x reference)
import functools

import jax
import jax.numpy as jnp
from jax import lax
from jax.experimental import pallas as pl
from jax.experimental.pallas import tpu as pltpu
from jax.experimental.pallas import tpu_sc as plsc

G = 2
V = 320
DG = 128
H = 512
BT = 8192
TOK_BLK = 2048
N_BLK = BT // TOK_BLK

NS = 16
RPW = BT // NS
CH = 128
NCH = RPW // CH
IDX_ROWS = BT // CH


def _proj_argmax_body(hs_ref, w_ref, b_ref, idx0_ref, idx1_ref, ppl_ref,
                      counts_scr):
    i = pl.program_id(0)

    @pl.when(i == 0)
    def _init():
        counts_scr[...] = jnp.zeros_like(counts_scr)

    hs = hs_ref[...]
    w = w_ref[...]
    iota_col = lax.broadcasted_iota(jnp.int32, (V, 2), 0).astype(jnp.float32)
    aux = jnp.where(lax.broadcasted_iota(jnp.int32, (V, 2), 1) == 0,
                    1.0, iota_col)
    ones_row = jnp.ones((8, TOK_BLK), jnp.float32)
    crows = []
    for g, idx_ref in ((0, idx0_ref), (1, idx1_ref)):
        wg = w[g * V:(g + 1) * V, :]
        logits = lax.dot_general(
            hs, wg, (((1,), (1,)), ((), ())),
            preferred_element_type=jnp.float32)
        logits = logits + b_ref[0, g * V:(g + 1) * V][None, :]
        m = jnp.max(logits, axis=1, keepdims=True)
        eqf = (logits == m).astype(jnp.float32)
        prods = lax.dot_general(eqf, aux, (((1,), (0,)), ((), ())),
                                precision=lax.Precision.HIGHEST,
                                preferred_element_type=jnp.float32)
        hist = lax.dot_general(ones_row, eqf, (((1,), (0,)), ((), ())),
                               precision=lax.Precision.HIGHEST,
                               preferred_element_type=jnp.float32)
        crows.append(hist[0:1, :])
        idx = prods[:, 1].astype(jnp.int32)
        idx_ref[...] = (idx + g * V).reshape(TOK_BLK // CH, CH)

        any_tie = jnp.max(prods[:, 0]) > 1.5

        @pl.when(any_tie)
        def _exact(logits=logits, m=m, idx_ref=idx_ref, g=g):
            iota = lax.broadcasted_iota(jnp.int32, logits.shape, 1)
            ex = jnp.min(jnp.where(logits == m, iota, V), axis=1)
            idx_ref[...] = (ex + g * V).reshape(TOK_BLK // CH, CH)
    counts_scr[...] += jnp.concatenate(crows, axis=0)

    @pl.when(i == N_BLK - 1)
    def _finish():
        p = counts_scr[...] * (1.0 / BT)
        ent = -jnp.sum(p * jnp.log(p + 1e-7), axis=1, keepdims=True)
        ppl_ref[...] = jnp.sum(jnp.exp(ent), axis=0, keepdims=True)


_proj_argmax = pl.pallas_call(
    _proj_argmax_body,
    grid=(N_BLK,),
    in_specs=[
        pl.BlockSpec((TOK_BLK, H), lambda i: (i, 0)),
        pl.BlockSpec((G * V, H), lambda i: (0, 0)),
        pl.BlockSpec((1, G * V), lambda i: (0, 0)),
    ],
    out_specs=[
        pl.BlockSpec((TOK_BLK // CH, CH), lambda i: (i, 0)),
        pl.BlockSpec((TOK_BLK // CH, CH), lambda i: (i, 0)),
        pl.BlockSpec((1, 1), lambda i: (0, 0)),
    ],
    out_shape=[
        jax.ShapeDtypeStruct((IDX_ROWS, CH), jnp.int32),
        jax.ShapeDtypeStruct((IDX_ROWS, CH), jnp.int32),
        jax.ShapeDtypeStruct((1, 1), jnp.float32),
    ],
    scratch_shapes=[pltpu.VMEM((G, V), jnp.float32)],
)


@functools.cache
def _make_sc_gather():
    mesh = plsc.VectorSubcoreMesh(core_axis_name="c", subcore_axis_name="s")

    @functools.partial(
        pl.kernel,
        mesh=mesh,
        out_type=jax.ShapeDtypeStruct((4, BT // 4, G * DG), jnp.float32),
        scratch_types=[
            pltpu.VMEM((NCH, CH), jnp.int32),
            pltpu.VMEM((RPW, DG), jnp.float32),
            pltpu.VMEM_SHARED((G * V, DG), jnp.float32),
            pltpu.SemaphoreType.DMA,
            pltpu.SemaphoreType.DMA,
        ],
    )
    def _sc_gather(table_hbm, idx0_hbm, idx1_hbm, out_hbm, idx_v, rows_v,
                   tab_sp, gsem, wsem):
        cid = lax.axis_index("c")
        sid = lax.axis_index("s")
        bat = sid // 4
        s0 = (sid % 4) * RPW

        @pl.when(sid == 0)
        def _stage():
            pltpu.sync_copy(table_hbm, tab_sp)

        plsc.subcore_barrier()

        def run(g, idx_hbm):
            pltpu.sync_copy(idx_hbm.at[pl.ds(sid * NCH, NCH)], idx_v)
            gathers = [
                pltpu.async_copy(tab_sp.at[idx_v.at[j]],
                                 rows_v.at[pl.ds(j * CH, CH)], gsem)
                for j in range(NCH)
            ]
            writes = []
            for j in range(NCH):
                gathers[j].wait()
                writes.append(pltpu.async_copy(
                    rows_v.at[pl.ds(j * CH, CH)],
                    out_hbm.at[bat, pl.ds(s0 + j * CH, CH),
                               pl.ds(g * DG, DG)], wsem))
            for wcp in writes:
                wcp.wait()

        @pl.when(cid == 0)
        def _g0():
            run(0, idx0_hbm)

        @pl.when(cid == 1)
        def _g1():
            run(1, idx1_hbm)

    return _sc_gather


def kernel(hidden_states, W, b, codevectors):
    batch, seq, hidden = hidden_states.shape
    hs2 = hidden_states.reshape(batch * seq, hidden)
    idx0, idx1, ppl = _proj_argmax(hs2, W, b.reshape(1, G * V))
    table = codevectors.reshape(G * V, DG)
    codevecs = _make_sc_gather()(table, idx0, idx1)
    return codevecs, ppl[0, 0]


# speedup vs baseline: 1.6637x; 1.6637x over previous
import functools

import jax
import jax.numpy as jnp
from jax import lax
from jax.experimental import pallas as pl
from jax.experimental.pallas import tpu as pltpu
from jax.experimental.pallas import tpu_sc as plsc

G = 2
V = 320
DG = 128
H = 512
BT = 8192
TOK_BLK = 2048
N_BLK = BT // TOK_BLK

NS = 16
RPW = BT // NS
CH = 128
NCH = RPW // CH
IDX_ROWS = BT // CH


def _proj_argmax_body(hs_ref, w_ref, b_ref, idx0_ref, idx1_ref, ppl_ref,
                      counts_scr):
    i = pl.program_id(0)

    @pl.when(i == 0)
    def _init():
        counts_scr[...] = jnp.zeros_like(counts_scr)

    hs = hs_ref[...]
    w = w_ref[...]
    crows = []
    for g, idx_ref in ((0, idx0_ref), (1, idx1_ref)):
        wg = w[g * V:(g + 1) * V, :]
        logits = lax.dot_general(
            hs, wg, (((1,), (1,)), ((), ())),
            preferred_element_type=jnp.float32)
        logits = logits + b_ref[0, g * V:(g + 1) * V][None, :]
        m = jnp.max(logits, axis=1, keepdims=True)
        eq = logits == m
        iota = lax.broadcasted_iota(jnp.int32, logits.shape, 1)
        idx = jnp.min(jnp.where(eq, iota, V), axis=1)
        crows.append(jnp.sum(eq.astype(jnp.float32), axis=0, keepdims=True))
        idx_ref[...] = (idx + g * V).reshape(TOK_BLK // CH, CH)
    counts_scr[...] += jnp.concatenate(crows, axis=0)

    @pl.when(i == N_BLK - 1)
    def _finish():
        p = counts_scr[...] * (1.0 / BT)
        ent = -jnp.sum(p * jnp.log(p + 1e-7), axis=1, keepdims=True)
        ppl_ref[...] = jnp.sum(jnp.exp(ent), axis=0, keepdims=True)


_proj_argmax = pl.pallas_call(
    _proj_argmax_body,
    grid=(N_BLK,),
    in_specs=[
        pl.BlockSpec((TOK_BLK, H), lambda i: (i, 0)),
        pl.BlockSpec((G * V, H), lambda i: (0, 0)),
        pl.BlockSpec((1, G * V), lambda i: (0, 0)),
    ],
    out_specs=[
        pl.BlockSpec((TOK_BLK // CH, CH), lambda i: (i, 0)),
        pl.BlockSpec((TOK_BLK // CH, CH), lambda i: (i, 0)),
        pl.BlockSpec((1, 1), lambda i: (0, 0)),
    ],
    out_shape=[
        jax.ShapeDtypeStruct((IDX_ROWS, CH), jnp.int32),
        jax.ShapeDtypeStruct((IDX_ROWS, CH), jnp.int32),
        jax.ShapeDtypeStruct((1, 1), jnp.float32),
    ],
    scratch_shapes=[pltpu.VMEM((G, V), jnp.float32)],
)


@functools.cache
def _make_sc_gather():
    mesh = plsc.VectorSubcoreMesh(core_axis_name="c", subcore_axis_name="s")

    @functools.partial(
        pl.kernel,
        mesh=mesh,
        out_type=jax.ShapeDtypeStruct((4, BT // 4, G * DG), jnp.float32),
        scratch_types=[
            pltpu.VMEM((NCH, CH), jnp.int32),
            pltpu.VMEM((RPW, DG), jnp.float32),
            pltpu.VMEM_SHARED((G * V, DG), jnp.float32),
            pltpu.SemaphoreType.DMA,
            pltpu.SemaphoreType.DMA,
        ],
    )
    def _sc_gather(table_hbm, idx0_hbm, idx1_hbm, out_hbm, idx_v, rows_v,
                   tab_sp, gsem, wsem):
        cid = lax.axis_index("c")
        sid = lax.axis_index("s")
        bat = sid // 4
        s0 = (sid % 4) * RPW

        @pl.when(sid == 0)
        def _stage():
            pltpu.sync_copy(table_hbm, tab_sp)

        plsc.subcore_barrier()

        def run(g, idx_hbm):
            pltpu.sync_copy(idx_hbm.at[pl.ds(sid * NCH, NCH)], idx_v)
            gathers = [
                pltpu.async_copy(tab_sp.at[idx_v.at[j]],
                                 rows_v.at[pl.ds(j * CH, CH)], gsem)
                for j in range(NCH)
            ]
            writes = []
            for j in range(NCH):
                gathers[j].wait()
                writes.append(pltpu.async_copy(
                    rows_v.at[pl.ds(j * CH, CH)],
                    out_hbm.at[bat, pl.ds(s0 + j * CH, CH),
                               pl.ds(g * DG, DG)], wsem))
            for wcp in writes:
                wcp.wait()

        @pl.when(cid == 0)
        def _g0():
            run(0, idx0_hbm)

        @pl.when(cid == 1)
        def _g1():
            run(1, idx1_hbm)

    return _sc_gather


def kernel(hidden_states, W, b, codevectors):
    batch, seq, hidden = hidden_states.shape
    hs2 = hidden_states.reshape(batch * seq, hidden)
    idx0, idx1, ppl = _proj_argmax(hs2, W, b.reshape(1, G * V))
    table = codevectors.reshape(G * V, DG)
    codevecs = _make_sc_gather()(table, idx0, idx1)
    return codevecs, ppl[0, 0]
